# baseline (speedup 1.0000x reference)
import jax
import jax.numpy as jnp
from jax.experimental import pallas as pl
from jax.experimental.pallas import tpu as pltpu
from jax.experimental.pallas import tpu_sc as plsc
from jax import lax

_NW = 32
_NBUF = 2
_L = 16


def kernel(indices, table):
    B, H = indices.shape
    V, D = table.shape
    idx_t = jnp.transpose(indices).astype(jnp.int32)
    per_w = B // _NW

    mesh = plsc.VectorSubcoreMesh(core_axis_name="core", subcore_axis_name="subcore")

    @pl.kernel(
        out_type=jax.ShapeDtypeStruct((H, D, B), table.dtype),
        mesh=mesh,
        compiler_params=pltpu.CompilerParams(
            use_tc_tiling_on_sc=False, needs_layout_passes=False),
        scratch_types=(
            [pltpu.VMEM((H, per_w), jnp.int32)]
            + [pltpu.VMEM((per_w, D), jnp.float32) for _ in range(_NBUF)]
            + [pltpu.VMEM((D, per_w), jnp.float32) for _ in range(_NBUF)]
            + [pltpu.SemaphoreType.DMA for _ in range(2 * _NBUF + 1)]
        ),
    )
    def gather_kernel(x_hbm, i_hbm, o_hbm, idx_v, *rest):
        bufs = rest[:_NBUF]
        tbufs = rest[_NBUF:2 * _NBUF]
        gsems = rest[2 * _NBUF:3 * _NBUF]
        wsems = rest[3 * _NBUF:4 * _NBUF]
        isem = rest[4 * _NBUF]
        wid = lax.axis_index("subcore") * 2 + lax.axis_index("core")
        base = wid * per_w
        pltpu.async_copy(i_hbm.at[:, pl.ds(base, per_w)], idx_v, isem).wait()

        def start_gather(h, b):
            pltpu.async_copy(x_hbm.at[idx_v.at[h]], bufs[b], gsems[b])

        for b in range(_NBUF):
            start_gather(b, b)

        @pl.loop(0, H, step=_NBUF)
        def _(r):
            for b in range(_NBUF):
                cur = r + b
                pltpu.make_async_copy(x_hbm.at[idx_v.at[0]], bufs[b], gsems[b]).wait()

                @pl.when(cur >= _NBUF)
                def _():
                    pltpu.make_async_copy(
                        tbufs[b], o_hbm.at[0, :, pl.ds(base, per_w)], wsems[b]).wait()

                @pl.loop(0, per_w // _L)
                def _(g):
                    rows = g * _L + jnp.arange(_L, dtype=jnp.int32)
                    for d in range(D):
                        cols = jnp.full((_L,), d, jnp.int32)
                        tbufs[b][d, pl.ds(g * _L, _L)] = plsc.load_gather(
                            bufs[b], [rows, cols])

                pltpu.async_copy(
                    tbufs[b], o_hbm.at[cur, :, pl.ds(base, per_w)], wsems[b])

                @pl.when(cur + _NBUF < H)
                def _():
                    start_gather(cur + _NBUF, b)

        for b in range(_NBUF):
            pltpu.make_async_copy(
                tbufs[b], o_hbm.at[0, :, pl.ds(base, per_w)], wsems[b]).wait()

    out_t = gather_kernel(table, idx_t)
    return jnp.transpose(out_t, (2, 0, 1))


# speedup vs baseline: 1.5285x; 1.5285x over previous
import jax
import jax.numpy as jnp
from jax.experimental import pallas as pl
from jax.experimental.pallas import tpu as pltpu
from jax.experimental.pallas import tpu_sc as plsc
from jax import lax

_NW = 32
_NBUF = 2
_L = 16


def kernel(indices, table):
    B, H = indices.shape
    V, D = table.shape
    idx_t = jnp.transpose(indices).astype(jnp.int32)
    per_w = B // _NW

    mesh = plsc.VectorSubcoreMesh(core_axis_name="core", subcore_axis_name="subcore")

    @pl.kernel(
        out_type=jax.ShapeDtypeStruct((H, D, B), table.dtype),
        mesh=mesh,
        compiler_params=pltpu.CompilerParams(
            use_tc_tiling_on_sc=False, needs_layout_passes=False),
        scratch_types=(
            [pltpu.VMEM((H, per_w), jnp.int32)]
            + [pltpu.VMEM((per_w, D), jnp.float32) for _ in range(_NBUF)]
            + [pltpu.VMEM((D, per_w + 1), jnp.float32) for _ in range(_NBUF)]
            + [pltpu.SemaphoreType.DMA for _ in range(2 * _NBUF + 1)]
        ),
    )
    def gather_kernel(x_hbm, i_hbm, o_hbm, idx_v, *rest):
        bufs = rest[:_NBUF]
        tbufs = rest[_NBUF:2 * _NBUF]
        gsems = rest[2 * _NBUF:3 * _NBUF]
        wsems = rest[3 * _NBUF:4 * _NBUF]
        isem = rest[4 * _NBUF]
        wid = lax.axis_index("subcore") * 2 + lax.axis_index("core")
        base = wid * per_w
        pltpu.async_copy(i_hbm.at[:, pl.ds(base, per_w)], idx_v, isem).wait()

        def start_gather(h, b):
            pltpu.async_copy(x_hbm.at[idx_v.at[h]], bufs[b], gsems[b])

        for b in range(_NBUF):
            start_gather(b, b)

        @pl.loop(0, H, step=_NBUF)
        def _(r):
            for b in range(_NBUF):
                cur = r + b
                pltpu.make_async_copy(x_hbm.at[idx_v.at[0]], bufs[b], gsems[b]).wait()

                @pl.when(cur >= _NBUF)
                def _():
                    pltpu.make_async_copy(
                        tbufs[b].at[:, pl.ds(0, per_w)],
                        o_hbm.at[0, :, pl.ds(base, per_w)], wsems[b]).wait()

                d_lo = jnp.arange(_L, dtype=jnp.int32)
                d_hi = d_lo + _L

                @pl.loop(0, per_w, step=8)
                def _(v0):
                    for j in range(8):
                        v = v0 + j
                        cols = jnp.full((_L,), v, jnp.int32)
                        plsc.store_scatter(
                            tbufs[b], [d_lo, cols], bufs[b][v, pl.ds(0, _L)])
                        plsc.store_scatter(
                            tbufs[b], [d_hi, cols], bufs[b][v, pl.ds(_L, _L)])

                pltpu.async_copy(
                    tbufs[b].at[:, pl.ds(0, per_w)],
                    o_hbm.at[cur, :, pl.ds(base, per_w)], wsems[b])

                @pl.when(cur + _NBUF < H)
                def _():
                    start_gather(cur + _NBUF, b)

        for b in range(_NBUF):
            pltpu.make_async_copy(
                tbufs[b].at[:, pl.ds(0, per_w)],
                o_hbm.at[0, :, pl.ds(base, per_w)], wsems[b]).wait()

    out_t = gather_kernel(table, idx_t)
    return jnp.transpose(out_t, (2, 0, 1))


# speedup vs baseline: 1.5312x; 1.0018x over previous
import jax
import jax.numpy as jnp
from jax.experimental import pallas as pl
from jax.experimental.pallas import tpu as pltpu
from jax.experimental.pallas import tpu_sc as plsc
from jax import lax

_NW = 32
_NBUF = 2
_L = 16


def kernel(indices, table):
    B, H = indices.shape
    V, D = table.shape
    idx_t = jnp.transpose(indices).astype(jnp.int32)
    per_w = B // _NW

    mesh = plsc.VectorSubcoreMesh(core_axis_name="core", subcore_axis_name="subcore")

    @pl.kernel(
        out_type=jax.ShapeDtypeStruct((H, D, B), table.dtype),
        mesh=mesh,
        compiler_params=pltpu.CompilerParams(
            use_tc_tiling_on_sc=False, needs_layout_passes=False),
        scratch_types=(
            [pltpu.VMEM((H, per_w), jnp.int32)]
            + [pltpu.VMEM((per_w, D), jnp.float32) for _ in range(_NBUF)]
            + [pltpu.VMEM((D, per_w + 1), jnp.float32) for _ in range(_NBUF)]
            + [pltpu.SemaphoreType.DMA for _ in range(2 * _NBUF + 1)]
        ),
    )
    def gather_kernel(x_hbm, i_hbm, o_hbm, idx_v, *rest):
        bufs = rest[:_NBUF]
        tbufs = rest[_NBUF:2 * _NBUF]
        gsems = rest[2 * _NBUF:3 * _NBUF]
        wsems = rest[3 * _NBUF:4 * _NBUF]
        isem = rest[4 * _NBUF]
        wid = lax.axis_index("subcore") * 2 + lax.axis_index("core")
        base = wid * per_w
        pltpu.async_copy(i_hbm.at[:, pl.ds(base, per_w)], idx_v, isem).wait()

        def start_gather(h, b):
            pltpu.async_copy(x_hbm.at[idx_v.at[h]], bufs[b], gsems[b])

        for b in range(_NBUF):
            start_gather(b, b)

        @pl.loop(0, H, step=_NBUF)
        def _(r):
            for b in range(_NBUF):
                cur = r + b
                pltpu.make_async_copy(x_hbm.at[idx_v.at[0]], bufs[b], gsems[b]).wait()

                @pl.when(cur >= _NBUF)
                def _():
                    pltpu.make_async_copy(
                        tbufs[b].at[:, pl.ds(0, per_w)],
                        o_hbm.at[0, :, pl.ds(base, per_w)], wsems[b]).wait()

                d_lo = jnp.arange(_L, dtype=jnp.int32)
                d_hi = d_lo + _L

                @pl.loop(0, per_w, step=16)
                def _(v0):
                    for j in range(16):
                        v = v0 + j
                        cols = jnp.full((_L,), v, jnp.int32)
                        plsc.store_scatter(
                            tbufs[b], [d_lo, cols], bufs[b][v, pl.ds(0, _L)])
                        plsc.store_scatter(
                            tbufs[b], [d_hi, cols], bufs[b][v, pl.ds(_L, _L)])

                pltpu.async_copy(
                    tbufs[b].at[:, pl.ds(0, per_w)],
                    o_hbm.at[cur, :, pl.ds(base, per_w)], wsems[b])

                @pl.when(cur + _NBUF < H)
                def _():
                    start_gather(cur + _NBUF, b)

        for b in range(_NBUF):
            pltpu.make_async_copy(
                tbufs[b].at[:, pl.ds(0, per_w)],
                o_hbm.at[0, :, pl.ds(base, per_w)], wsems[b]).wait()

    out_t = gather_kernel(table, idx_t)
    return jnp.transpose(out_t, (2, 0, 1))
